# baseline (speedup 1.0000x reference)
import functools
from functools import partial

import jax
import jax.numpy as jnp
from jax import lax
from jax.experimental import pallas as pl
from jax.experimental.pallas import tpu as pltpu
from jax.experimental.pallas import tpu_sc as plsc

B, T, Kt, C = 4, 12, 3, 64
T1 = T - Kt + 1
T2 = T1 - Kt + 1
LANES = 128
NTILES = 16
EPAD = NTILES * LANES


def _dot(a, b):
    return lax.dot_general(a, b, (((a.ndim - 1,), (0,)), ((), ())),
                           preferred_element_type=jnp.float32)


def _t1_body(x_ref, w_ref, b_ref, o_ref):
    for t in range(T1):
        s = jnp.broadcast_to(b_ref[0][None, :], (x_ref.shape[3], 2 * C))
        for k in range(Kt):
            xk = x_ref[0, :, t + k, :]
            s = s + lax.dot_general(xk, w_ref[k], (((0,), (0,)), ((), ())),
                                    preferred_element_type=jnp.float32)
        o_ref[t, 0] = s[:, :C] * jax.nn.sigmoid(s[:, C:])


def _t1_call(x, w, b, nblk):
    _, _, _, N = x.shape
    grid = (B, pl.cdiv(N, nblk))
    return pl.pallas_call(
        _t1_body,
        grid=grid,
        in_specs=[
            pl.BlockSpec((1, C, T, nblk), lambda bb, nn: (bb, 0, 0, nn)),
            pl.BlockSpec((Kt, C, 2 * C), lambda bb, nn: (0, 0, 0)),
            pl.BlockSpec((1, 2 * C), lambda bb, nn: (0, 0)),
        ],
        out_specs=pl.BlockSpec((T1, 1, nblk, C), lambda bb, nn: (0, bb, nn, 0)),
        out_shape=jax.ShapeDtypeStruct((T1, B, N, C), jnp.float32),
    )(x, w, b)


def _rup(n, m):
    return -(-n // m) * m


def _sage_body(agg_ref, cnt_ref, xd_ref, wl_ref, bl_ref, wr_ref, g_ref, o_ref):
    cnt = jnp.maximum(cnt_ref[:, 0:1], 1.0)
    mean = agg_ref[0] * (1.0 / cnt)
    xd = xd_ref[...]
    h = _dot(mean, wl_ref[...]) + bl_ref[...] + _dot(xd, wr_ref[...])
    h = jnp.maximum(h, 0.0)
    a = jax.nn.sigmoid(g_ref[0, 0])
    o_ref[...] = (1.0 - a) * xd + a * h


def _sage_call(agg3, cnt, xd, wl, bl, wr, g, ndst, rt):
    nb = ndst // rt
    return pl.pallas_call(
        _sage_body,
        grid=(T1, nb),
        in_specs=[
            pl.BlockSpec((1, rt, C), lambda t, r: (t, r, 0)),
            pl.BlockSpec((rt, 16), lambda t, r: (r, 0)),
            pl.BlockSpec((rt, C), lambda t, r: (t * nb + r, 0)),
            pl.BlockSpec((C, C), lambda t, r: (0, 0)),
            pl.BlockSpec((1, C), lambda t, r: (0, 0)),
            pl.BlockSpec((C, C), lambda t, r: (0, 0)),
            pl.BlockSpec((1, 1), lambda t, r: (0, 0)),
        ],
        out_specs=pl.BlockSpec((rt, C), lambda t, r: (t * nb + r, 0)),
        out_shape=jax.ShapeDtypeStruct((T1 * ndst, C), jnp.float32),
    )(agg3, cnt, xd, wl, bl, wr, g)


def _room_pre_body(agg_ref, cnt_ref, room_ref, tf_ref, of_ref, dh_ref, dv_ref,
                   wl_ref, bl_ref, wr_ref, tw_ref, tb_ref, ow_ref, ob_ref,
                   ghw_ref, gvw_ref, gs_ref, yh_ref, yv_ref):
    cnt = jnp.maximum(cnt_ref[:, 0:1], 1.0)
    mean = agg_ref[0] * (1.0 / cnt)
    room0 = room_ref[...]
    h = _dot(mean, wl_ref[...]) + bl_ref[...] + _dot(room0, wr_ref[...])
    h = jnp.maximum(h, 0.0)
    a = jax.nn.sigmoid(gs_ref[0, 0])
    r = (1.0 - a) * room0 + a * h
    tf = _dot(tf_ref[0], tw_ref[...]) + tb_ref[...]
    a = jax.nn.sigmoid(gs_ref[0, 1])
    r = (1.0 - a) * r + a * tf
    of = _dot(of_ref[0], ow_ref[...]) + ob_ref[...]
    a = jax.nn.sigmoid(gs_ref[0, 2])
    r = (1.0 - a) * r + a * of
    dh = lax.rsqrt(dh_ref[:, 0:1] + 1.0)
    dv = lax.rsqrt(dv_ref[:, 0:1] + 1.0)
    yh_ref[...] = _dot(r, ghw_ref[...]) * dh
    yv_ref[...] = _dot(r, gvw_ref[...]) * dv


def _room_pre_call(agg3, cnt, room0, tf, of, degh, degv,
                   wl, bl, wr, tw, tb, ow, ob, ghw, gvw, gs, ndst, rt,
                   out_rows):
    nb = ndst // rt
    bpr = (ndst // B) // rt
    mat = pl.BlockSpec((C, C), lambda t, r: (0, 0))
    vec = pl.BlockSpec((1, C), lambda t, r: (0, 0))
    big = pl.BlockSpec((rt, C), lambda t, r: (t * nb + r, 0))
    cntspec = pl.BlockSpec((rt, 16), lambda t, r: (r, 0))
    return pl.pallas_call(
        _room_pre_body,
        grid=(T1, nb),
        in_specs=[
            pl.BlockSpec((1, rt, C), lambda t, r: (t, r, 0)),
            cntspec, big,
            pl.BlockSpec((1, 1, C), lambda t, r: (t * B + r // bpr, 0, 0)),
            pl.BlockSpec((1, 1, C), lambda t, r: (t * B + r // bpr, 0, 0)),
            cntspec, cntspec,
            mat, vec, mat, mat, vec, mat, vec, mat, mat,
            pl.BlockSpec((1, 3), lambda t, r: (0, 0)),
        ],
        out_specs=[big, big],
        out_shape=[jax.ShapeDtypeStruct((out_rows, C), jnp.float32)] * 2,
    )(agg3, cnt, room0, tf, of, degh, degv,
      wl, bl, wr, tw, tb, ow, ob, ghw, gvw, gs)


def _room_post_body(ah_ref, av_ref, yh_ref, yv_ref, dh_ref, dv_ref,
                    bh_ref, bv_ref, o_ref):
    dh = lax.rsqrt(dh_ref[:, 0:1] + 1.0)
    dv = lax.rsqrt(dv_ref[:, 0:1] + 1.0)
    o = dh * (ah_ref[0] + yh_ref[...]) + bh_ref[...]
    o = o + dv * (av_ref[0] + yv_ref[...]) + bv_ref[...]
    o_ref[...] = jnp.maximum(o, 0.0)


def _room_post_call(ah3, av3, yh, yv, degh, degv, bh, bv, ndst, rt):
    nb = ndst // rt
    big = pl.BlockSpec((rt, C), lambda t, r: (t * nb + r, 0))
    agspec = pl.BlockSpec((1, rt, C), lambda t, r: (t, r, 0))
    cntspec = pl.BlockSpec((rt, 16), lambda t, r: (r, 0))
    vec = pl.BlockSpec((1, C), lambda t, r: (0, 0))
    return pl.pallas_call(
        _room_post_body,
        grid=(T1, nb),
        in_specs=[agspec, agspec, big, big, cntspec, cntspec, vec, vec],
        out_specs=big,
        out_shape=jax.ShapeDtypeStruct((T1 * ndst, C), jnp.float32),
    )(ah3, av3, yh, yv, degh, degv, bh, bv)


def _t2_body(x_ref, w_ref, b_ref, g_ref, be_ref, o_ref):
    for t in range(T2):
        s = jnp.broadcast_to(b_ref[0][None, :], (x_ref.shape[2], 2 * C))
        for k in range(Kt):
            s = s + _dot(x_ref[t + k, 0], w_ref[k])
        gl = s[:, :C] * jax.nn.sigmoid(s[:, C:])
        mu = jnp.mean(gl, axis=-1, keepdims=True)
        var = jnp.mean((gl - mu) * (gl - mu), axis=-1, keepdims=True)
        xn = (gl - mu) * lax.rsqrt(var + 1e-5) * g_ref[...] + be_ref[...]
        o_ref[0, :, t, :] = xn.T


def _t2_call(xf, w, b, g, be, nblk):
    N = xf.shape[2]
    grid = (B, pl.cdiv(N, nblk))
    return pl.pallas_call(
        _t2_body,
        grid=grid,
        in_specs=[
            pl.BlockSpec((T1, 1, nblk, C), lambda bb, nn: (0, bb, nn, 0)),
            pl.BlockSpec((Kt, C, 2 * C), lambda bb, nn: (0, 0, 0)),
            pl.BlockSpec((1, 2 * C), lambda bb, nn: (0, 0)),
            pl.BlockSpec((1, C), lambda bb, nn: (0, 0)),
            pl.BlockSpec((1, C), lambda bb, nn: (0, 0)),
        ],
        out_specs=pl.BlockSpec((1, C, T2, nblk), lambda bb, nn: (bb, 0, 0, nn)),
        out_shape=jax.ShapeDtypeStruct((B, C, T2, N), jnp.float32),
    )(xf, w, b, g, be)


def _mesh():
    return plsc.VectorSubcoreMesh(core_axis_name="c", subcore_axis_name="s",
                                  num_cores=2, num_subcores=NTILES)


_SC_PARAMS = pltpu.CompilerParams(use_tc_tiling_on_sc=False)


def _seg_kernel_body(src_hbm, dst_hbm, x_hbm, offs_hbm, agg_hbm,
                     agg_sp, idx_sb, idx_db, rows, zbuf, offv, stepv,
                     gsem, ssem, *, ndst, depth):
    s = lax.axis_index("s")
    c = lax.axis_index("c")
    nb = src_hbm.shape[0]
    my_chunks = nb // NTILES
    groups = my_chunks // depth
    ndstp = _rup(ndst + 1, 2048)
    stripe = ndstp // NTILES
    zrows = zbuf.shape[0]
    nz = stripe // zrows

    def zero_z(i, _):
        for j in range(4):
            zbuf[i, pl.ds(16 * j, 16)] = jnp.zeros((16,), jnp.float32)
        return 0
    lax.fori_loop(0, zrows, zero_z, 0)

    pltpu.sync_copy(src_hbm.at[pl.ds(s * my_chunks, my_chunks)], idx_sb)
    pltpu.sync_copy(dst_hbm.at[pl.ds(s * my_chunks, my_chunks)], idx_db)
    pltpu.sync_copy(offs_hbm.at[pl.ds(c, 1)], offv)
    pltpu.sync_copy(offs_hbm.at[pl.ds(2, 1)], stepv)
    for j in range(my_chunks):
        for i in range(LANES // 16):
            sl = pl.ds(16 * i, 16)
            idx_sb[j, sl] = idx_sb[j, sl] + offv[0]

    def t_body(kk, _):
        for z in range(nz):
            pltpu.sync_copy(zbuf,
                            agg_sp.at[pl.ds(s * stripe + z * zrows, zrows)])
        plsc.subcore_barrier()
        gd = [pltpu.async_copy(x_hbm.at[idx_sb.at[d]], rows.at[0, d],
                               gsem.at[0, d]) for d in range(depth)]
        sd = [None] * depth
        for g in range(groups):
            p = g % 2
            for d in range(depth):
                gd[d].wait()
            if g >= 1:
                for d in range(depth):
                    sd[d].wait()
            if g + 1 < groups:
                gd = [pltpu.async_copy(x_hbm.at[idx_sb.at[(g + 1) * depth + d]],
                                       rows.at[1 - p, d], gsem.at[1 - p, d])
                      for d in range(depth)]
            sd = [pltpu.async_copy(rows.at[p, d],
                                   agg_sp.at[idx_db.at[g * depth + d]],
                                   ssem.at[p, d], add=True)
                  for d in range(depth)]
        for d in range(depth):
            sd[d].wait()
        for j in range(my_chunks):
            for i in range(LANES // 16):
                sl = pl.ds(16 * i, 16)
                idx_sb[j, sl] = idx_sb[j, sl] + stepv[0]
        plsc.subcore_barrier()
        t = 2 * kk + c
        pltpu.sync_copy(agg_sp.at[pl.ds(s * stripe, stripe)],
                        agg_hbm.at[t, pl.ds(s * stripe, stripe)])
        plsc.subcore_barrier()
        return 0
    lax.fori_loop(0, T1 // 2, t_body, 0)


@functools.lru_cache(maxsize=None)
def _seg_kernel(nb, xrows, ndst, depth, zrows):
    ndstp = _rup(ndst + 1, 2048)
    my_chunks = nb // NTILES
    return pl.kernel(
        partial(_seg_kernel_body, ndst=ndst, depth=depth),
        out_type=jax.ShapeDtypeStruct((T1, ndstp, C), jnp.float32),
        mesh=_mesh(),
        compiler_params=_SC_PARAMS,
        scratch_types=[
            pltpu.VMEM_SHARED((ndstp, C), jnp.float32),
            pltpu.VMEM((my_chunks, LANES), jnp.int32),
            pltpu.VMEM((my_chunks, LANES), jnp.int32),
            pltpu.VMEM((2, depth, LANES, C), jnp.float32),
            pltpu.VMEM((zrows, C), jnp.float32),
            pltpu.VMEM((1, 16), jnp.int32),
            pltpu.VMEM((1, 16), jnp.int32),
            pltpu.SemaphoreType.DMA((2, depth)),
            pltpu.SemaphoreType.DMA((2, depth)),
        ],
    )


def _seg_call(src2d, dst2d, x, ndst, sstride, depth):
    zrows = 64
    offs = jnp.broadcast_to(
        (jnp.arange(3, dtype=jnp.int32) * sstride)[:, None], (3, 16))
    kfn = _seg_kernel(src2d.shape[0], x.shape[0], ndst, depth, zrows)
    return kfn(src2d, dst2d, x, offs)


_HSIZES = None


def _hist_body(d1, d2, d3, d4, o1, o2, o3, o4,
               hsp, ones_v, idx_d, zbuf, *, sizes):
    s = lax.axis_index("s")
    c = lax.axis_index("c")

    def zrow(i, _):
        zbuf[i, :] = jnp.zeros((16,), jnp.float32)
        return 0
    lax.fori_loop(0, zbuf.shape[0], zrow, 0)

    def orow(i, _):
        ones_v[i, :] = jnp.ones((16,), jnp.float32)
        return 0
    lax.fori_loop(0, LANES, orow, 0)

    ins = [d1, d2, d3, d4]
    outs = [o1, o2, o3, o4]
    halves = [sz // 2 for sz in sizes]
    for r in range(4):
        half = halves[r]
        stripe = half // NTILES
        zr = zbuf.shape[0]
        for z in range(stripe // zr):
            pltpu.sync_copy(zbuf.at[pl.ds(0, zr)],
                            hsp.at[pl.ds(s * stripe + z * zr, zr)])
        rem = stripe % zr
        if rem:
            pltpu.sync_copy(zbuf.at[pl.ds(0, rem)],
                            hsp.at[pl.ds(s * stripe + stripe - rem, rem)])
        plsc.subcore_barrier()
        nb = ins[r].shape[0]
        my_chunks = nb // NTILES

        def e_body(j, _, r=r, half=half):
            row = s * my_chunks + j
            pltpu.sync_copy(ins[r].at[pl.ds(row, 1)], idx_d)
            hv = jnp.full((16,), half, jnp.int32)
            bv = jnp.full((16,), c * half, jnp.int32)
            disc = hv + lax.iota(jnp.int32, 16)
            for i in range(LANES // 16):
                sl = pl.ds(16 * i, 16)
                d = idx_d[0, sl] - bv
                ok = (d >= 0) & (d < hv)
                idx_d[0, sl] = jnp.where(ok, d, disc)
            pltpu.sync_copy(ones_v, hsp.at[idx_d.at[0]], add=True)
            return 0
        lax.fori_loop(0, my_chunks, e_body, 0)
        plsc.subcore_barrier()
        pltpu.sync_copy(hsp.at[pl.ds(s * stripe, stripe)],
                        outs[r].at[pl.ds(c * half + s * stripe, stripe)])
        plsc.subcore_barrier()


def _hist_call(d1, d2, d3, d4, n1, n2, n3, n4):
    sizes = tuple(_rup(n + 1, 128) for n in (n1, n2, n3, n4))
    kfn = pl.kernel(
        partial(_hist_body, sizes=sizes),
        out_type=[jax.ShapeDtypeStruct((sz, 16), jnp.float32) for sz in sizes],
        mesh=_mesh(),
        compiler_params=_SC_PARAMS,
        scratch_types=[
            pltpu.VMEM_SHARED((sizes[0] // 2 + 16, 16), jnp.float32),
            pltpu.VMEM((LANES, 16), jnp.float32),
            pltpu.VMEM((1, LANES), jnp.int32),
            pltpu.VMEM((max(sizes) // 2 // NTILES // 4, 16), jnp.float32),
        ],
    )
    return kfn(d1, d2, d3, d4)


def _prep_w(W):
    Wm = jnp.transpose(W[..., 0], (2, 1, 0))
    add = jnp.concatenate([jnp.eye(C, dtype=W.dtype),
                           jnp.zeros((C, C), W.dtype)], axis=1)
    return Wm.at[Kt - 1].add(add)


def _prep_edges(ei, ndst, nbt):
    E = ei.shape[1]
    Epad = nbt * LANES
    spread = _rup(ndst + 1, 2048) - ndst
    pad_dst = ndst + jnp.arange(Epad - E, dtype=jnp.int32) % spread
    src = jnp.concatenate([ei[0], jnp.zeros((Epad - E,), jnp.int32)])
    dst = jnp.concatenate([ei[1], pad_dst])
    return src.reshape(-1, LANES), dst.reshape(-1, LANES)


def kernel(xs, edges, params):
    NN = {nt: xs[nt].shape[3] for nt in xs}
    ND = {nt: B * NN[nt] for nt in xs}

    w1 = {nt: _prep_w(params['t1_W_' + nt]) for nt in NN}
    w2 = {nt: _prep_w(params['t2_W_' + nt]) for nt in NN}

    def b2d(v):
        return v.reshape(1, -1)

    flat1 = {}
    for nt in NN:
        nblk = min(512, NN[nt])
        flat1[nt] = _t1_call(xs[nt], w1[nt], b2d(params['t1_b_' + nt]), nblk)

    prop = flat1['property'].reshape(-1, C)
    dev0 = flat1['device'].reshape(-1, C)
    room0 = flat1['room'].reshape(-1, C)
    timef = flat1['time'].reshape(-1, C)
    outf = flat1['outside'].reshape(-1, C)

    sp2d, dp2d = _prep_edges(edges['p2d'], ND['device'], 320)
    sd2r, dd2r = _prep_edges(edges['d2r'], ND['room'], 160)
    sh, dh = _prep_edges(edges['room_h'], ND['room'], 160)
    sv, dv = _prep_edges(edges['room_v'], ND['room'], 160)

    cnt1, cnt2, degh, degv = _hist_call(
        dp2d, dd2r, dh, dv, ND['device'], ND['room'], ND['room'], ND['room'])

    RT = 1000
    RX = T1 * ND['device']
    agg1 = _seg_call(sp2d, dp2d, prop, ND['device'], ND['property'], 2)
    dev1 = _sage_call(agg1, cnt1, dev0, params['sage1_Wl'],
                      b2d(params['sage1_bl']), params['sage1_Wr'],
                      params['g_p2d_dev'].reshape(1, 1), ND['device'], RT)

    agg2 = _seg_call(sd2r, dd2r, dev1, ND['room'], ND['device'], 2)
    gs = jnp.stack([params['g_d2r_room'], params['g_time2room'],
                    params['g_outside2room']]).reshape(1, 3)
    yh, yv = _room_pre_call(
        agg2, cnt2, room0, timef[:, None, :], outf[:, None, :], degh, degv,
        params['sage2_Wl'], b2d(params['sage2_bl']), params['sage2_Wr'],
        params['time_W'], b2d(params['time_b']),
        params['out_W'], b2d(params['out_b']),
        params['gcnh_W'], params['gcnv_W'], gs, ND['room'], RT, RX)

    aggh = _seg_call(sh, dh, yh, ND['room'], ND['room'], 2)
    aggv = _seg_call(sv, dv, yv, ND['room'], ND['room'], 2)
    room4 = _room_post_call(aggh, aggv, yh, yv, degh, degv,
                            b2d(params['gcnh_b']), b2d(params['gcnv_b']),
                            ND['room'], RT)

    flat2 = {
        'property': flat1['property'],
        'device': dev1.reshape(T1, B, NN['device'], C),
        'room': room4.reshape(T1, B, NN['room'], C),
        'time': flat1['time'],
        'outside': flat1['outside'],
    }
    out = {}
    for nt in NN:
        nblk = min(512, NN[nt])
        out[nt] = _t2_call(flat2[nt], w2[nt], b2d(params['t2_b_' + nt]),
                           b2d(params['ln_g_' + nt]), b2d(params['ln_b_' + nt]),
                           nblk)
    return out


# speedup vs baseline: 1.7387x; 1.7387x over previous
import functools
from functools import partial

import jax
import jax.numpy as jnp
from jax import lax
from jax.experimental import pallas as pl
from jax.experimental.pallas import tpu as pltpu
from jax.experimental.pallas import tpu_sc as plsc

B, T, Kt, C = 4, 12, 3, 64
T1 = T - Kt + 1
T2 = T1 - Kt + 1
LANES = 128
NTILES = 16
EPAD = NTILES * LANES


def _dot(a, b):
    return lax.dot_general(a, b, (((a.ndim - 1,), (0,)), ((), ())),
                           preferred_element_type=jnp.float32)


def _t1_body(x_ref, w_ref, b_ref, o_ref):
    for t in range(T1):
        s = jnp.broadcast_to(b_ref[0][None, :], (x_ref.shape[3], 2 * C))
        for k in range(Kt):
            xk = x_ref[0, :, t + k, :]
            s = s + lax.dot_general(xk, w_ref[k], (((0,), (0,)), ((), ())),
                                    preferred_element_type=jnp.float32)
        o_ref[t, 0] = s[:, :C] * jax.nn.sigmoid(s[:, C:])


def _t1_call(x, w, b, nblk):
    _, _, _, N = x.shape
    grid = (B, pl.cdiv(N, nblk))
    return pl.pallas_call(
        _t1_body,
        grid=grid,
        in_specs=[
            pl.BlockSpec((1, C, T, nblk), lambda bb, nn: (bb, 0, 0, nn)),
            pl.BlockSpec((Kt, C, 2 * C), lambda bb, nn: (0, 0, 0)),
            pl.BlockSpec((1, 2 * C), lambda bb, nn: (0, 0)),
        ],
        out_specs=pl.BlockSpec((T1, 1, nblk, C), lambda bb, nn: (0, bb, nn, 0)),
        out_shape=jax.ShapeDtypeStruct((T1, B, N, C), jnp.float32),
    )(x, w, b)


def _rup(n, m):
    return -(-n // m) * m


def _sage_body(agg_ref, cnt_ref, xd_ref, wl_ref, bl_ref, wr_ref, g_ref, o_ref):
    cnt = jnp.maximum(cnt_ref[:, 0:1], 1.0)
    mean = agg_ref[0] * (1.0 / cnt)
    xd = xd_ref[...]
    h = _dot(mean, wl_ref[...]) + bl_ref[...] + _dot(xd, wr_ref[...])
    h = jnp.maximum(h, 0.0)
    a = jax.nn.sigmoid(g_ref[0, 0])
    o_ref[...] = (1.0 - a) * xd + a * h


def _sage_call(agg3, cnt, xd, wl, bl, wr, g, ndst, rt):
    nb = ndst // rt
    return pl.pallas_call(
        _sage_body,
        grid=(T1, nb),
        in_specs=[
            pl.BlockSpec((1, rt, C), lambda t, r: (t, r, 0)),
            pl.BlockSpec((rt, 16), lambda t, r: (r, 0)),
            pl.BlockSpec((rt, C), lambda t, r: (t * nb + r, 0)),
            pl.BlockSpec((C, C), lambda t, r: (0, 0)),
            pl.BlockSpec((1, C), lambda t, r: (0, 0)),
            pl.BlockSpec((C, C), lambda t, r: (0, 0)),
            pl.BlockSpec((1, 1), lambda t, r: (0, 0)),
        ],
        out_specs=pl.BlockSpec((rt, C), lambda t, r: (t * nb + r, 0)),
        out_shape=jax.ShapeDtypeStruct((T1 * ndst, C), jnp.float32),
    )(agg3, cnt, xd, wl, bl, wr, g)


def _room_pre_body(agg_ref, cnt_ref, room_ref, tf_ref, of_ref, dh_ref, dv_ref,
                   wl_ref, bl_ref, wr_ref, tw_ref, tb_ref, ow_ref, ob_ref,
                   ghw_ref, gvw_ref, gs_ref, yh_ref, yv_ref):
    cnt = jnp.maximum(cnt_ref[:, 0:1], 1.0)
    mean = agg_ref[0] * (1.0 / cnt)
    room0 = room_ref[...]
    h = _dot(mean, wl_ref[...]) + bl_ref[...] + _dot(room0, wr_ref[...])
    h = jnp.maximum(h, 0.0)
    a = jax.nn.sigmoid(gs_ref[0, 0])
    r = (1.0 - a) * room0 + a * h
    tf = _dot(tf_ref[0], tw_ref[...]) + tb_ref[...]
    a = jax.nn.sigmoid(gs_ref[0, 1])
    r = (1.0 - a) * r + a * tf
    of = _dot(of_ref[0], ow_ref[...]) + ob_ref[...]
    a = jax.nn.sigmoid(gs_ref[0, 2])
    r = (1.0 - a) * r + a * of
    dh = lax.rsqrt(dh_ref[:, 0:1] + 1.0)
    dv = lax.rsqrt(dv_ref[:, 0:1] + 1.0)
    yh_ref[...] = _dot(r, ghw_ref[...]) * dh
    yv_ref[...] = _dot(r, gvw_ref[...]) * dv


def _room_pre_call(agg3, cnt, room0, tf, of, degh, degv,
                   wl, bl, wr, tw, tb, ow, ob, ghw, gvw, gs, ndst, rt,
                   out_rows):
    nb = ndst // rt
    bpr = (ndst // B) // rt
    mat = pl.BlockSpec((C, C), lambda t, r: (0, 0))
    vec = pl.BlockSpec((1, C), lambda t, r: (0, 0))
    big = pl.BlockSpec((rt, C), lambda t, r: (t * nb + r, 0))
    cntspec = pl.BlockSpec((rt, 16), lambda t, r: (r, 0))
    return pl.pallas_call(
        _room_pre_body,
        grid=(T1, nb),
        in_specs=[
            pl.BlockSpec((1, rt, C), lambda t, r: (t, r, 0)),
            cntspec, big,
            pl.BlockSpec((1, 1, C), lambda t, r: (t * B + r // bpr, 0, 0)),
            pl.BlockSpec((1, 1, C), lambda t, r: (t * B + r // bpr, 0, 0)),
            cntspec, cntspec,
            mat, vec, mat, mat, vec, mat, vec, mat, mat,
            pl.BlockSpec((1, 3), lambda t, r: (0, 0)),
        ],
        out_specs=[big, big],
        out_shape=[jax.ShapeDtypeStruct((out_rows, C), jnp.float32)] * 2,
    )(agg3, cnt, room0, tf, of, degh, degv,
      wl, bl, wr, tw, tb, ow, ob, ghw, gvw, gs)


def _room_post_body(ah_ref, av_ref, yh_ref, yv_ref, dh_ref, dv_ref,
                    bh_ref, bv_ref, o_ref):
    dh = lax.rsqrt(dh_ref[:, 0:1] + 1.0)
    dv = lax.rsqrt(dv_ref[:, 0:1] + 1.0)
    o = dh * (ah_ref[0] + yh_ref[...]) + bh_ref[...]
    o = o + dv * (av_ref[0] + yv_ref[...]) + bv_ref[...]
    o_ref[...] = jnp.maximum(o, 0.0)


def _room_post_call(ah3, av3, yh, yv, degh, degv, bh, bv, ndst, rt):
    nb = ndst // rt
    big = pl.BlockSpec((rt, C), lambda t, r: (t * nb + r, 0))
    agspec = pl.BlockSpec((1, rt, C), lambda t, r: (t, r, 0))
    cntspec = pl.BlockSpec((rt, 16), lambda t, r: (r, 0))
    vec = pl.BlockSpec((1, C), lambda t, r: (0, 0))
    return pl.pallas_call(
        _room_post_body,
        grid=(T1, nb),
        in_specs=[agspec, agspec, big, big, cntspec, cntspec, vec, vec],
        out_specs=big,
        out_shape=jax.ShapeDtypeStruct((T1 * ndst, C), jnp.float32),
    )(ah3, av3, yh, yv, degh, degv, bh, bv)


def _t2_body(x_ref, w_ref, b_ref, g_ref, be_ref, o_ref):
    for t in range(T2):
        s = jnp.broadcast_to(b_ref[0][None, :], (x_ref.shape[2], 2 * C))
        for k in range(Kt):
            s = s + _dot(x_ref[t + k, 0], w_ref[k])
        gl = s[:, :C] * jax.nn.sigmoid(s[:, C:])
        mu = jnp.mean(gl, axis=-1, keepdims=True)
        var = jnp.mean((gl - mu) * (gl - mu), axis=-1, keepdims=True)
        xn = (gl - mu) * lax.rsqrt(var + 1e-5) * g_ref[...] + be_ref[...]
        o_ref[0, :, t, :] = xn.T


def _t2_call(xf, w, b, g, be, nblk):
    N = xf.shape[2]
    grid = (B, pl.cdiv(N, nblk))
    return pl.pallas_call(
        _t2_body,
        grid=grid,
        in_specs=[
            pl.BlockSpec((T1, 1, nblk, C), lambda bb, nn: (0, bb, nn, 0)),
            pl.BlockSpec((Kt, C, 2 * C), lambda bb, nn: (0, 0, 0)),
            pl.BlockSpec((1, 2 * C), lambda bb, nn: (0, 0)),
            pl.BlockSpec((1, C), lambda bb, nn: (0, 0)),
            pl.BlockSpec((1, C), lambda bb, nn: (0, 0)),
        ],
        out_specs=pl.BlockSpec((1, C, T2, nblk), lambda bb, nn: (bb, 0, 0, nn)),
        out_shape=jax.ShapeDtypeStruct((B, C, T2, N), jnp.float32),
    )(xf, w, b, g, be)


def _mesh():
    return plsc.VectorSubcoreMesh(core_axis_name="c", subcore_axis_name="s",
                                  num_cores=2, num_subcores=NTILES)


_SC_PARAMS = pltpu.CompilerParams(use_tc_tiling_on_sc=False)


def _seg_kernel_body(src_hbm, dst_hbm, x_hbm, offs_hbm, agg_hbm,
                     agg_sp, idx_sb, idx_db, rows, zbuf, offv, stepv,
                     gsem, ssem, *, ndst, depth):
    s = lax.axis_index("s")
    c = lax.axis_index("c")
    nb = src_hbm.shape[0]
    my_chunks = nb // NTILES
    groups = my_chunks // depth
    ndstp = _rup(ndst + 1, 2048)
    stripe = ndstp // NTILES
    zrows = zbuf.shape[0]
    nz = stripe // zrows

    def zero_z(i, _):
        for j in range(4):
            zbuf[i, pl.ds(16 * j, 16)] = jnp.zeros((16,), jnp.float32)
        return 0
    lax.fori_loop(0, zrows, zero_z, 0)

    pltpu.sync_copy(src_hbm.at[pl.ds(s * my_chunks, my_chunks)], idx_sb)
    pltpu.sync_copy(dst_hbm.at[pl.ds(s * my_chunks, my_chunks)], idx_db)
    pltpu.sync_copy(offs_hbm.at[pl.ds(c, 1)], offv)
    pltpu.sync_copy(offs_hbm.at[pl.ds(2, 1)], stepv)
    for j in range(my_chunks):
        for i in range(LANES // 16):
            sl = pl.ds(16 * i, 16)
            idx_sb[j, sl] = idx_sb[j, sl] + offv[0]

    def t_body(kk, _):
        for z in range(nz):
            pltpu.sync_copy(zbuf,
                            agg_sp.at[pl.ds(s * stripe + z * zrows, zrows)])
        plsc.subcore_barrier()
        gd = [pltpu.async_copy(x_hbm.at[idx_sb.at[d]], rows.at[0, d],
                               gsem.at[0, d]) for d in range(depth)]
        sd = [None] * depth
        for g in range(groups):
            p = g % 2
            for d in range(depth):
                gd[d].wait()
            if g >= 1:
                for d in range(depth):
                    sd[d].wait()
            if g + 1 < groups:
                gd = [pltpu.async_copy(x_hbm.at[idx_sb.at[(g + 1) * depth + d]],
                                       rows.at[1 - p, d], gsem.at[1 - p, d])
                      for d in range(depth)]
            sd = [pltpu.async_copy(rows.at[p, d],
                                   agg_sp.at[idx_db.at[g * depth + d]],
                                   ssem.at[p, d], add=True)
                  for d in range(depth)]
        for d in range(depth):
            sd[d].wait()
        for j in range(my_chunks):
            for i in range(LANES // 16):
                sl = pl.ds(16 * i, 16)
                idx_sb[j, sl] = idx_sb[j, sl] + stepv[0]
        plsc.subcore_barrier()
        t = 2 * kk + c
        pltpu.sync_copy(agg_sp.at[pl.ds(s * stripe, stripe)],
                        agg_hbm.at[t, pl.ds(s * stripe, stripe)])
        plsc.subcore_barrier()
        return 0
    lax.fori_loop(0, T1 // 2, t_body, 0)


@functools.lru_cache(maxsize=None)
def _seg_kernel(nb, xrows, ndst, depth, zrows):
    ndstp = _rup(ndst + 1, 2048)
    my_chunks = nb // NTILES
    return pl.kernel(
        partial(_seg_kernel_body, ndst=ndst, depth=depth),
        out_type=jax.ShapeDtypeStruct((T1, ndstp, C), jnp.float32),
        mesh=_mesh(),
        compiler_params=_SC_PARAMS,
        scratch_types=[
            pltpu.VMEM_SHARED((ndstp, C), jnp.float32),
            pltpu.VMEM((my_chunks, LANES), jnp.int32),
            pltpu.VMEM((my_chunks, LANES), jnp.int32),
            pltpu.VMEM((2, depth, LANES, C), jnp.float32),
            pltpu.VMEM((zrows, C), jnp.float32),
            pltpu.VMEM((1, 16), jnp.int32),
            pltpu.VMEM((1, 16), jnp.int32),
            pltpu.SemaphoreType.DMA((2, depth)),
            pltpu.SemaphoreType.DMA((2, depth)),
        ],
    )


def _seg_call(src2d, dst2d, x, ndst, sstride, depth):
    zrows = 64
    offs = jnp.broadcast_to(
        (jnp.arange(3, dtype=jnp.int32) * sstride)[:, None], (3, 16))
    kfn = _seg_kernel(src2d.shape[0], x.shape[0], ndst, depth, zrows)
    return kfn(src2d, dst2d, x, offs)


_HSIZES = None


def _hist_body(d1, d2, d3, d4, o1, o2, o3, o4,
               hsp, ones_v, idx_d, zbuf, *, sizes):
    s = lax.axis_index("s")
    c = lax.axis_index("c")

    def zrow(i, _):
        zbuf[i, :] = jnp.zeros((16,), jnp.float32)
        return 0
    lax.fori_loop(0, zbuf.shape[0], zrow, 0)

    def orow(i, _):
        ones_v[i, :] = jnp.ones((16,), jnp.float32)
        return 0
    lax.fori_loop(0, LANES, orow, 0)

    ins = [d1, d2, d3, d4]
    outs = [o1, o2, o3, o4]
    halves = [sz // 2 for sz in sizes]
    for r in range(4):
        half = halves[r]
        stripe = half // NTILES
        zr = zbuf.shape[0]
        for z in range(stripe // zr):
            pltpu.sync_copy(zbuf.at[pl.ds(0, zr)],
                            hsp.at[pl.ds(s * stripe + z * zr, zr)])
        rem = stripe % zr
        if rem:
            pltpu.sync_copy(zbuf.at[pl.ds(0, rem)],
                            hsp.at[pl.ds(s * stripe + stripe - rem, rem)])
        plsc.subcore_barrier()
        nb = ins[r].shape[0]
        my_chunks = nb // NTILES

        def e_body(j, _, r=r, half=half):
            row = s * my_chunks + j
            pltpu.sync_copy(ins[r].at[pl.ds(row, 1)], idx_d)
            hv = jnp.full((16,), half, jnp.int32)
            bv = jnp.full((16,), c * half, jnp.int32)
            disc = hv + lax.iota(jnp.int32, 16)
            for i in range(LANES // 16):
                sl = pl.ds(16 * i, 16)
                d = idx_d[0, sl] - bv
                ok = (d >= 0) & (d < hv)
                idx_d[0, sl] = jnp.where(ok, d, disc)
            pltpu.sync_copy(ones_v, hsp.at[idx_d.at[0]], add=True)
            return 0
        lax.fori_loop(0, my_chunks, e_body, 0)
        plsc.subcore_barrier()
        pltpu.sync_copy(hsp.at[pl.ds(s * stripe, stripe)],
                        outs[r].at[pl.ds(c * half + s * stripe, stripe)])
        plsc.subcore_barrier()


def _hist_call(d1, d2, d3, d4, n1, n2, n3, n4):
    sizes = tuple(_rup(n + 1, 128) for n in (n1, n2, n3, n4))
    kfn = pl.kernel(
        partial(_hist_body, sizes=sizes),
        out_type=[jax.ShapeDtypeStruct((sz, 16), jnp.float32) for sz in sizes],
        mesh=_mesh(),
        compiler_params=_SC_PARAMS,
        scratch_types=[
            pltpu.VMEM_SHARED((sizes[0] // 2 + 16, 16), jnp.float32),
            pltpu.VMEM((LANES, 16), jnp.float32),
            pltpu.VMEM((1, LANES), jnp.int32),
            pltpu.VMEM((max(sizes) // 2 // NTILES // 4, 16), jnp.float32),
        ],
    )
    return kfn(d1, d2, d3, d4)


def _prep_w(W):
    Wm = jnp.transpose(W[..., 0], (2, 1, 0))
    add = jnp.concatenate([jnp.eye(C, dtype=W.dtype),
                           jnp.zeros((C, C), W.dtype)], axis=1)
    return Wm.at[Kt - 1].add(add)


def _prep_edges(ei, ndst, nsrc, nbt):
    E = ei.shape[1]
    Epad = nbt * LANES
    spread = _rup(ndst + 1, 2048) - ndst
    pad = jnp.arange(Epad - E, dtype=jnp.int32)
    src = jnp.concatenate([ei[0], pad % nsrc])
    dst = jnp.concatenate([ei[1], ndst + pad % spread])
    return src.reshape(-1, LANES), dst.reshape(-1, LANES)


def kernel(xs, edges, params):
    NN = {nt: xs[nt].shape[3] for nt in xs}
    ND = {nt: B * NN[nt] for nt in xs}

    w1 = {nt: _prep_w(params['t1_W_' + nt]) for nt in NN}
    w2 = {nt: _prep_w(params['t2_W_' + nt]) for nt in NN}

    def b2d(v):
        return v.reshape(1, -1)

    flat1 = {}
    for nt in NN:
        nblk = min(512, NN[nt])
        flat1[nt] = _t1_call(xs[nt], w1[nt], b2d(params['t1_b_' + nt]), nblk)

    prop = flat1['property'].reshape(-1, C)
    dev0 = flat1['device'].reshape(-1, C)
    room0 = flat1['room'].reshape(-1, C)
    timef = flat1['time'].reshape(-1, C)
    outf = flat1['outside'].reshape(-1, C)

    sp2d, dp2d = _prep_edges(edges['p2d'], ND['device'], ND['property'], 320)
    sd2r, dd2r = _prep_edges(edges['d2r'], ND['room'], ND['device'], 160)
    sh, dh = _prep_edges(edges['room_h'], ND['room'], ND['room'], 160)
    sv, dv = _prep_edges(edges['room_v'], ND['room'], ND['room'], 160)

    cnt1, cnt2, degh, degv = _hist_call(
        dp2d, dd2r, dh, dv, ND['device'], ND['room'], ND['room'], ND['room'])

    RT = 1000
    RX = T1 * ND['device']
    agg1 = _seg_call(sp2d, dp2d, prop, ND['device'], ND['property'], 2)
    dev1 = _sage_call(agg1, cnt1, dev0, params['sage1_Wl'],
                      b2d(params['sage1_bl']), params['sage1_Wr'],
                      params['g_p2d_dev'].reshape(1, 1), ND['device'], RT)

    agg2 = _seg_call(sd2r, dd2r, dev1, ND['room'], ND['device'], 2)
    gs = jnp.stack([params['g_d2r_room'], params['g_time2room'],
                    params['g_outside2room']]).reshape(1, 3)
    yh, yv = _room_pre_call(
        agg2, cnt2, room0, timef[:, None, :], outf[:, None, :], degh, degv,
        params['sage2_Wl'], b2d(params['sage2_bl']), params['sage2_Wr'],
        params['time_W'], b2d(params['time_b']),
        params['out_W'], b2d(params['out_b']),
        params['gcnh_W'], params['gcnv_W'], gs, ND['room'], RT, RX)

    aggh = _seg_call(sh, dh, yh, ND['room'], ND['room'], 2)
    aggv = _seg_call(sv, dv, yv, ND['room'], ND['room'], 2)
    room4 = _room_post_call(aggh, aggv, yh, yv, degh, degv,
                            b2d(params['gcnh_b']), b2d(params['gcnv_b']),
                            ND['room'], RT)

    flat2 = {
        'property': flat1['property'],
        'device': dev1.reshape(T1, B, NN['device'], C),
        'room': room4.reshape(T1, B, NN['room'], C),
        'time': flat1['time'],
        'outside': flat1['outside'],
    }
    out = {}
    for nt in NN:
        nblk = min(512, NN[nt])
        out[nt] = _t2_call(flat2[nt], w2[nt], b2d(params['t2_b_' + nt]),
                           b2d(params['ln_g_' + nt]), b2d(params['ln_b_' + nt]),
                           nblk)
    return out
